# baseline (speedup 1.0000x reference)
import functools

import jax
import jax.numpy as jnp
from jax import lax
from jax.experimental import pallas as pl
from jax.experimental.pallas import tpu as pltpu
from jax.experimental.pallas import tpu_sc as plsc

NC = 2
NS = 16
NW = NC * NS
LANES = 16


def _sc_edge_body(n, h, epw, nch, k,
                  u_hbm, src_hbm, dst_hbm, ea_hbm, out_hbm,
                  src_v, dst_v, ea_v, rows0_v, rows1_v, msg_v, nbuf_v,
                  acc_sh, sem0, sem1):
    c = lax.axis_index("c")
    s = lax.axis_index("s")
    wid = s * NC + c
    npt = (n // NS) // 8 * 8
    tail = n - NS * npt

    if True:
        def zrow(i, _):
            nbuf_v[i, :] = jnp.zeros((LANES,), jnp.float32)
            return 0
        lax.fori_loop(0, npt, zrow, 0)
        pltpu.sync_copy(nbuf_v, acc_sh.at[pl.ds(s * npt, npt)])
        if tail:
            @pl.when(s == 0)
            def _():
                pltpu.sync_copy(nbuf_v.at[pl.ds(0, tail)],
                                acc_sh.at[pl.ds(NS * npt, tail)])

        rpt = epw // k
        grp = epw // LANES
        pltpu.sync_copy(src_hbm.at[pl.ds(wid * rpt, rpt)], src_v)
        pltpu.sync_copy(dst_hbm.at[pl.ds(wid * rpt, rpt)], dst_v)
        for d in range(ea_v.shape[0]):
            pltpu.sync_copy(ea_hbm.at[d].at[pl.ds(wid * grp, grp)],
                            ea_v.at[d])
        plsc.subcore_barrier()

        def compute_and_scatter(rows_v, j):
            @plsc.parallel_loop(0, k // LANES, unroll=4)
            def _(g):
                row = j * (k // LANES) + g
                a0 = ea_v[0, row, :]
                a1 = ea_v[1, row, :]
                a2 = ea_v[2, row, :]
                a3 = ea_v[3, row, :]
                for t in range(LANES):
                    i = g * LANES + t
                    r0 = rows_v[i, pl.ds(0, LANES)]
                    r1 = rows_v[i, pl.ds(LANES, LANES)]
                    r2 = rows_v[i, pl.ds(2 * LANES, LANES)]
                    r3 = rows_v[i, pl.ds(3 * LANES, LANES)]
                    rb = rows_v[i, pl.ds(4 * LANES, LANES)]
                    msg_v[i, :] = (rb + a0[t] * r0 + a1[t] * r1
                                   + a2[t] * r2 + a3[t] * r3)
            pltpu.sync_copy(msg_v, acc_sh.at[dst_v.at[j]], add=True)

        pltpu.async_copy(u_hbm.at[src_v.at[0]], rows0_v, sem0)

        def pair(j2, _):
            j = 2 * j2
            pltpu.async_copy(u_hbm.at[src_v.at[j + 1]], rows1_v, sem1)
            pltpu.make_async_copy(u_hbm.at[src_v.at[j]], rows0_v, sem0).wait()
            compute_and_scatter(rows0_v, j)

            @pl.when(j2 + 1 < nch // 2)
            def _():
                pltpu.async_copy(u_hbm.at[src_v.at[j + 2]], rows0_v, sem0)
            pltpu.make_async_copy(u_hbm.at[src_v.at[j + 1]], rows1_v,
                                  sem1).wait()
            compute_and_scatter(rows1_v, j + 1)
            return 0
        lax.fori_loop(0, nch // 2, pair, 0)

        plsc.subcore_barrier()
        pltpu.sync_copy(acc_sh.at[pl.ds(s * npt, npt)], nbuf_v)
        pltpu.sync_copy(nbuf_v, out_hbm.at[c].at[pl.ds(s * npt, npt)])
        if tail:
            @pl.when(s == 0)
            def _():
                pltpu.sync_copy(acc_sh.at[pl.ds(NS * npt, tail)],
                                nbuf_v.at[pl.ds(0, tail)])
                pltpu.sync_copy(nbuf_v.at[pl.ds(0, tail)],
                                out_hbm.at[c].at[pl.ds(NS * npt, tail)])


def _make_sc_aggregate(n, h, ed, ep, true_e):
    epw = ep // NW
    k = 128
    nch = epw // k
    assert epw * NW == ep and nch * k == epw and nch % 2 == 0
    uw = (ed + 1) * h
    mesh = plsc.VectorSubcoreMesh(core_axis_name="c", subcore_axis_name="s",
                                  num_cores=NC, num_subcores=NS)
    return pl.kernel(
        functools.partial(_sc_edge_body, n, h, epw, nch, k),
        out_type=jax.ShapeDtypeStruct((NC, n, h), jnp.float32),
        mesh=mesh,
        scratch_types=[
            pltpu.VMEM((nch, k), jnp.int32),
            pltpu.VMEM((nch, k), jnp.int32),
            pltpu.VMEM((ed, epw // LANES, LANES), jnp.float32),
            pltpu.VMEM((k, uw), jnp.float32),
            pltpu.VMEM((k, uw), jnp.float32),
            pltpu.VMEM((k, h), jnp.float32),
            pltpu.VMEM(((n // NS) // 8 * 8, h), jnp.float32),
            pltpu.VMEM_SHARED((n + max(ep - true_e, 8), h), jnp.float32),
            pltpu.SemaphoreType.DMA,
            pltpu.SemaphoreType.DMA,
        ],
        compiler_params=pltpu.CompilerParams(use_tc_tiling_on_sc=False),
    )


def _t1_body(x_ref, wemb_ref, bemb_ref, wcat_ref, h_ref, u_ref):
    hv = jnp.maximum(
        jnp.dot(x_ref[...], wemb_ref[...],
                preferred_element_type=jnp.float32) + bemb_ref[...], 0.0)
    h_ref[...] = hv
    u_ref[...] = jnp.dot(hv, wcat_ref[...], preferred_element_type=jnp.float32)


def _t2_body(acc_ref, h_ref, root_ref, bias_ref, wcat_ref, h_out_ref, u_ref):
    hv = jnp.maximum(
        acc_ref[0] + acc_ref[1]
        + jnp.dot(h_ref[...], root_ref[...],
                  preferred_element_type=jnp.float32) + bias_ref[...], 0.0)
    h_out_ref[...] = hv
    u_ref[...] = jnp.dot(hv, wcat_ref[...], preferred_element_type=jnp.float32)


def _t3_body(g, acc_ref, h_ref, root_ref, bias_ref, batch_ref, w1_ref,
             b1_ref, out_ref, pooled_ref):
    i = pl.program_id(0)
    hv = jnp.maximum(
        acc_ref[0] + acc_ref[1]
        + jnp.dot(h_ref[...], root_ref[...],
                  preferred_element_type=jnp.float32) + bias_ref[...], 0.0)
    bn = h_ref.shape[0]
    gid = lax.broadcasted_iota(jnp.int32, (g, bn), 0)
    onehot = (gid == batch_ref[0]).astype(jnp.float32)
    contrib = jnp.dot(onehot, hv, preferred_element_type=jnp.float32)

    @pl.when(i == 0)
    def _():
        pooled_ref[...] = jnp.zeros_like(pooled_ref)

    pooled_ref[...] += contrib

    @pl.when(i == pl.num_programs(0) - 1)
    def _():
        out_ref[...] = jnp.dot(pooled_ref[...], w1_ref[...],
                               preferred_element_type=jnp.float32) + b1_ref[...]


def _wcat(nnW, nnb, ed, h):
    w = nnW.reshape(ed, h, h).transpose(1, 0, 2).reshape(h, ed * h)
    return jnp.concatenate([w, nnb.reshape(h, h)], axis=1)


def kernel(x, edge_index, edge_attr, batch, W_emb, b_emb, nnW1, nnb1, root1,
           bias1, nnW2, nnb2, root2, bias2, nnW3, nnb3, root3, bias3, W1, b1):
    n, d = x.shape
    h = W_emb.shape[1]
    e = edge_index.shape[1]
    ed = edge_attr.shape[1]
    o = W1.shape[1]
    g = 64
    uw = (ed + 1) * h
    bn = 1000
    nblk = n // bn
    k = 128
    ep = -(-e // (NW * 2 * k)) * (NW * 2 * k)
    pad = ep - e

    src3 = jnp.pad(edge_index[0], (0, pad)).reshape(ep // k, k)
    dst3 = jnp.concatenate(
        [edge_index[1], n + jnp.arange(pad, dtype=jnp.int32)]).reshape(
            ep // k, k)
    ea_t = jnp.pad(edge_attr.T, ((0, 0), (0, pad))).reshape(
        ed, ep // LANES, LANES)
    batch3 = batch.reshape(nblk, 1, bn)

    wc1 = _wcat(nnW1, nnb1, ed, h)
    wc2 = _wcat(nnW2, nnb2, ed, h)
    wc3 = _wcat(nnW3, nnb3, ed, h)

    sc_aggr = _make_sc_aggregate(n, h, ed, ep, e)

    full = lambda shape: pl.BlockSpec(shape, lambda i: tuple(0 for _ in shape))
    rows = lambda w: pl.BlockSpec((bn, w), lambda i: (i, 0))

    t1 = pl.pallas_call(
        _t1_body,
        grid=(nblk,),
        in_specs=[rows(d), full((d, h)), full((1, h)), full((h, uw))],
        out_specs=[rows(h), rows(uw)],
        out_shape=[jax.ShapeDtypeStruct((n, h), jnp.float32),
                   jax.ShapeDtypeStruct((n, uw), jnp.float32)],
    )

    t2 = pl.pallas_call(
        _t2_body,
        grid=(nblk,),
        in_specs=[pl.BlockSpec((NC, bn, h), lambda i: (0, i, 0)),
                  rows(h), full((h, h)), full((1, h)), full((h, uw))],
        out_specs=[rows(h), rows(uw)],
        out_shape=[jax.ShapeDtypeStruct((n, h), jnp.float32),
                   jax.ShapeDtypeStruct((n, uw), jnp.float32)],
    )

    t3 = pl.pallas_call(
        functools.partial(_t3_body, g),
        grid=(nblk,),
        in_specs=[pl.BlockSpec((NC, bn, h), lambda i: (0, i, 0)),
                  rows(h), full((h, h)), full((1, h)),
                  pl.BlockSpec((1, 1, bn), lambda i: (i, 0, 0)),
                  full((h, o)), full((1, o))],
        out_specs=pl.BlockSpec((g, o), lambda i: (0, 0)),
        out_shape=jax.ShapeDtypeStruct((g, o), jnp.float32),
        scratch_shapes=[pltpu.VMEM((g, h), jnp.float32)],
    )

    b_emb2 = b_emb.reshape(1, h)
    h0, u1 = t1(x, W_emb, b_emb2, wc1)
    acc1 = sc_aggr(u1, src3, dst3, ea_t)
    h1, u2 = t2(acc1, h0, root1, bias1.reshape(1, h), wc2)
    acc2 = sc_aggr(u2, src3, dst3, ea_t)
    h2, u3 = t2(acc2, h1, root2, bias2.reshape(1, h), wc3)
    acc3 = sc_aggr(u3, src3, dst3, ea_t)
    return t3(acc3, h2, root3, bias3.reshape(1, h), batch3, W1,
              b1.reshape(1, o))


# speedup vs baseline: 2.2617x; 2.2617x over previous
import functools

import jax
import jax.numpy as jnp
from jax import lax
from jax.experimental import pallas as pl
from jax.experimental.pallas import tpu as pltpu
from jax.experimental.pallas import tpu_sc as plsc

NC = 2
NS = 16
NW = NC * NS
LANES = 16


def _sc_edge_body(n, h, epw, nch, k,
                  u_hbm, src_hbm, dst_hbm, ea_hbm, out_hbm,
                  src_v, dst_v, ea_v, rows0_v, rows1_v, msg_v, nbuf_v,
                  acc_sh, sem0, sem1):
    c = lax.axis_index("c")
    s = lax.axis_index("s")
    wid = s * NC + c
    npt = (n // NS) // 8 * 8
    tail = n - NS * npt

    if True:
        def zrow(i, _):
            nbuf_v[i, :] = jnp.zeros((LANES,), jnp.float32)
            return 0
        lax.fori_loop(0, npt, zrow, 0)
        pltpu.sync_copy(nbuf_v, acc_sh.at[pl.ds(s * npt, npt)])
        if tail:
            @pl.when(s == 0)
            def _():
                pltpu.sync_copy(nbuf_v.at[pl.ds(0, tail)],
                                acc_sh.at[pl.ds(NS * npt, tail)])

        rpt = epw // k
        grp = epw // LANES
        pltpu.sync_copy(src_hbm.at[pl.ds(wid * rpt, rpt)], src_v)
        pltpu.sync_copy(dst_hbm.at[pl.ds(wid * rpt, rpt)], dst_v)
        for d in range(ea_v.shape[0]):
            pltpu.sync_copy(ea_hbm.at[d].at[pl.ds(wid * grp, grp)],
                            ea_v.at[d])
        plsc.subcore_barrier()

        def compute_and_scatter(rows_v, j):
            @plsc.parallel_loop(0, k // LANES, unroll=4)
            def _(g):
                row = j * (k // LANES) + g
                a0 = ea_v[0, row, :]
                a1 = ea_v[1, row, :]
                a2 = ea_v[2, row, :]
                a3 = ea_v[3, row, :]
                for t in range(LANES):
                    i = g * LANES + t
                    r0 = rows_v[i, pl.ds(0, LANES)]
                    r1 = rows_v[i, pl.ds(LANES, LANES)]
                    r2 = rows_v[i, pl.ds(2 * LANES, LANES)]
                    r3 = rows_v[i, pl.ds(3 * LANES, LANES)]
                    rb = rows_v[i, pl.ds(4 * LANES, LANES)]
                    msg_v[i, :] = (rb + a0[t] * r0 + a1[t] * r1
                                   + a2[t] * r2 + a3[t] * r3)
            pltpu.sync_copy(msg_v, acc_sh.at[dst_v.at[j]], add=True)

        pltpu.async_copy(u_hbm.at[src_v.at[0]], rows0_v, sem0)

        def pair(j2, _):
            j = 2 * j2
            pltpu.async_copy(u_hbm.at[src_v.at[j + 1]], rows1_v, sem1)
            pltpu.make_async_copy(u_hbm.at[src_v.at[j]], rows0_v, sem0).wait()
            compute_and_scatter(rows0_v, j)

            @pl.when(j2 + 1 < nch // 2)
            def _():
                pltpu.async_copy(u_hbm.at[src_v.at[j + 2]], rows0_v, sem0)
            pltpu.make_async_copy(u_hbm.at[src_v.at[j + 1]], rows1_v,
                                  sem1).wait()
            compute_and_scatter(rows1_v, j + 1)
            return 0
        lax.fori_loop(0, nch // 2, pair, 0)

        plsc.subcore_barrier()
        pltpu.sync_copy(acc_sh.at[pl.ds(s * npt, npt)], nbuf_v)
        pltpu.sync_copy(nbuf_v, out_hbm.at[c].at[pl.ds(s * npt, npt)])
        if tail:
            @pl.when(s == 0)
            def _():
                pltpu.sync_copy(acc_sh.at[pl.ds(NS * npt, tail)],
                                nbuf_v.at[pl.ds(0, tail)])
                pltpu.sync_copy(nbuf_v.at[pl.ds(0, tail)],
                                out_hbm.at[c].at[pl.ds(NS * npt, tail)])


def _make_sc_aggregate(n, h, ed, ep, true_e):
    epw = ep // NW
    k = 128
    nch = epw // k
    assert epw * NW == ep and nch * k == epw and nch % 2 == 0
    uw = (ed + 1) * h
    mesh = plsc.VectorSubcoreMesh(core_axis_name="c", subcore_axis_name="s",
                                  num_cores=NC, num_subcores=NS)
    return pl.kernel(
        functools.partial(_sc_edge_body, n, h, epw, nch, k),
        out_type=jax.ShapeDtypeStruct((NC, n, h), jnp.float32),
        mesh=mesh,
        scratch_types=[
            pltpu.VMEM((nch, k), jnp.int32),
            pltpu.VMEM((nch, k), jnp.int32),
            pltpu.VMEM((ed, epw // LANES, LANES), jnp.float32),
            pltpu.VMEM((k, uw), jnp.float32),
            pltpu.VMEM((k, uw), jnp.float32),
            pltpu.VMEM((k, h), jnp.float32),
            pltpu.VMEM(((n // NS) // 8 * 8, h), jnp.float32),
            pltpu.VMEM_SHARED((n + max(ep - true_e, 8), h), jnp.float32),
            pltpu.SemaphoreType.DMA,
            pltpu.SemaphoreType.DMA,
        ],
        compiler_params=pltpu.CompilerParams(use_tc_tiling_on_sc=False),
    )


def _t1_body(x_ref, wemb_ref, bemb_ref, wcat_ref, h_ref, u_ref):
    hv = jnp.maximum(
        jnp.dot(x_ref[...], wemb_ref[...],
                preferred_element_type=jnp.float32) + bemb_ref[...], 0.0)
    h_ref[...] = hv
    u_ref[...] = jnp.dot(hv, wcat_ref[...], preferred_element_type=jnp.float32)


def _t2_body(acc_ref, h_ref, root_ref, bias_ref, wcat_ref, h_out_ref, u_ref):
    hv = jnp.maximum(
        acc_ref[0] + acc_ref[1]
        + jnp.dot(h_ref[...], root_ref[...],
                  preferred_element_type=jnp.float32) + bias_ref[...], 0.0)
    h_out_ref[...] = hv
    u_ref[...] = jnp.dot(hv, wcat_ref[...], preferred_element_type=jnp.float32)


def _t3_body(g, acc_ref, h_ref, root_ref, bias_ref, batch_ref, w1_ref,
             b1_ref, out_ref, pooled_ref):
    i = pl.program_id(0)
    hv = jnp.maximum(
        acc_ref[0] + acc_ref[1]
        + jnp.dot(h_ref[...], root_ref[...],
                  preferred_element_type=jnp.float32) + bias_ref[...], 0.0)
    bn = h_ref.shape[0]
    gid = lax.broadcasted_iota(jnp.int32, (g, bn), 0)
    onehot = (gid == batch_ref[0]).astype(jnp.float32)
    contrib = jnp.dot(onehot, hv, preferred_element_type=jnp.float32)

    @pl.when(i == 0)
    def _():
        pooled_ref[...] = jnp.zeros_like(pooled_ref)

    pooled_ref[...] += contrib

    @pl.when(i == pl.num_programs(0) - 1)
    def _():
        out_ref[...] = jnp.dot(pooled_ref[...], w1_ref[...],
                               preferred_element_type=jnp.float32) + b1_ref[...]


def _wcat(nnW, nnb, ed, h):
    w = nnW.reshape(ed, h, h).transpose(1, 0, 2).reshape(h, ed * h)
    return jnp.concatenate([w, nnb.reshape(h, h)], axis=1)


def kernel(x, edge_index, edge_attr, batch, W_emb, b_emb, nnW1, nnb1, root1,
           bias1, nnW2, nnb2, root2, bias2, nnW3, nnb3, root3, bias3, W1, b1):
    n, d = x.shape
    h = W_emb.shape[1]
    e = edge_index.shape[1]
    ed = edge_attr.shape[1]
    o = W1.shape[1]
    g = 64
    uw = (ed + 1) * h
    bn = 1000
    nblk = n // bn
    k = 128
    ep = -(-e // (NW * 2 * k)) * (NW * 2 * k)
    pad = ep - e

    src3 = jnp.concatenate(
        [edge_index[0],
         jnp.arange(pad, dtype=jnp.int32) % n]).reshape(ep // k, k)
    dst3 = jnp.concatenate(
        [edge_index[1], n + jnp.arange(pad, dtype=jnp.int32)]).reshape(
            ep // k, k)
    ea_t = jnp.pad(edge_attr.T, ((0, 0), (0, pad))).reshape(
        ed, ep // LANES, LANES)
    batch3 = batch.reshape(nblk, 1, bn)

    wc1 = _wcat(nnW1, nnb1, ed, h)
    wc2 = _wcat(nnW2, nnb2, ed, h)
    wc3 = _wcat(nnW3, nnb3, ed, h)

    sc_aggr = _make_sc_aggregate(n, h, ed, ep, e)

    full = lambda shape: pl.BlockSpec(shape, lambda i: tuple(0 for _ in shape))
    rows = lambda w: pl.BlockSpec((bn, w), lambda i: (i, 0))

    t1 = pl.pallas_call(
        _t1_body,
        grid=(nblk,),
        in_specs=[rows(d), full((d, h)), full((1, h)), full((h, uw))],
        out_specs=[rows(h), rows(uw)],
        out_shape=[jax.ShapeDtypeStruct((n, h), jnp.float32),
                   jax.ShapeDtypeStruct((n, uw), jnp.float32)],
    )

    t2 = pl.pallas_call(
        _t2_body,
        grid=(nblk,),
        in_specs=[pl.BlockSpec((NC, bn, h), lambda i: (0, i, 0)),
                  rows(h), full((h, h)), full((1, h)), full((h, uw))],
        out_specs=[rows(h), rows(uw)],
        out_shape=[jax.ShapeDtypeStruct((n, h), jnp.float32),
                   jax.ShapeDtypeStruct((n, uw), jnp.float32)],
    )

    t3 = pl.pallas_call(
        functools.partial(_t3_body, g),
        grid=(nblk,),
        in_specs=[pl.BlockSpec((NC, bn, h), lambda i: (0, i, 0)),
                  rows(h), full((h, h)), full((1, h)),
                  pl.BlockSpec((1, 1, bn), lambda i: (i, 0, 0)),
                  full((h, o)), full((1, o))],
        out_specs=pl.BlockSpec((g, o), lambda i: (0, 0)),
        out_shape=jax.ShapeDtypeStruct((g, o), jnp.float32),
        scratch_shapes=[pltpu.VMEM((g, h), jnp.float32)],
    )

    b_emb2 = b_emb.reshape(1, h)
    h0, u1 = t1(x, W_emb, b_emb2, wc1)
    acc1 = sc_aggr(u1, src3, dst3, ea_t)
    h1, u2 = t2(acc1, h0, root1, bias1.reshape(1, h), wc2)
    acc2 = sc_aggr(u2, src3, dst3, ea_t)
    h2, u3 = t2(acc2, h1, root2, bias2.reshape(1, h), wc3)
    acc3 = sc_aggr(u3, src3, dst3, ea_t)
    return t3(acc3, h2, root3, bias3.reshape(1, h), batch3, W1,
              b1.reshape(1, o))


# speedup vs baseline: 2.3150x; 1.0236x over previous
import functools

import jax
import jax.numpy as jnp
from jax import lax
from jax.experimental import pallas as pl
from jax.experimental.pallas import tpu as pltpu
from jax.experimental.pallas import tpu_sc as plsc

NC = 2
NS = 16
NW = NC * NS
LANES = 16


def _sc_edge_body(n, h, epw, nch, k,
                  u_hbm, src_hbm, dst_hbm, ea_hbm, out_hbm,
                  src_v, dst_v, ea_v, rows0_v, rows1_v, msg0_v, msg1_v,
                  nbuf_v, acc_sh, sem0, sem1, sem2, sem3):
    c = lax.axis_index("c")
    s = lax.axis_index("s")
    wid = s * NC + c
    npt = (n // NS) // 8 * 8
    tail = n - NS * npt

    if True:
        def zrow(i, _):
            nbuf_v[i, :] = jnp.zeros((LANES,), jnp.float32)
            return 0
        lax.fori_loop(0, npt, zrow, 0)
        pltpu.sync_copy(nbuf_v, acc_sh.at[pl.ds(s * npt, npt)])
        if tail:
            @pl.when(s == 0)
            def _():
                pltpu.sync_copy(nbuf_v.at[pl.ds(0, tail)],
                                acc_sh.at[pl.ds(NS * npt, tail)])

        rpt = epw // k
        grp = epw // LANES
        pltpu.sync_copy(src_hbm.at[pl.ds(wid * rpt, rpt)], src_v)
        pltpu.sync_copy(dst_hbm.at[pl.ds(wid * rpt, rpt)], dst_v)
        for d in range(ea_v.shape[0]):
            pltpu.sync_copy(ea_hbm.at[d].at[pl.ds(wid * grp, grp)],
                            ea_v.at[d])
        plsc.subcore_barrier()

        def compute(rows_v, msg_v, j):
            @plsc.parallel_loop(0, k // LANES, unroll=4)
            def _(g):
                row = j * (k // LANES) + g
                a0 = ea_v[0, row, :]
                a1 = ea_v[1, row, :]
                a2 = ea_v[2, row, :]
                a3 = ea_v[3, row, :]
                for t in range(LANES):
                    i = g * LANES + t
                    r0 = rows_v[i, pl.ds(0, LANES)]
                    r1 = rows_v[i, pl.ds(LANES, LANES)]
                    r2 = rows_v[i, pl.ds(2 * LANES, LANES)]
                    r3 = rows_v[i, pl.ds(3 * LANES, LANES)]
                    rb = rows_v[i, pl.ds(4 * LANES, LANES)]
                    msg_v[i, :] = (rb + a0[t] * r0 + a1[t] * r1
                                   + a2[t] * r2 + a3[t] * r3)

        pltpu.async_copy(u_hbm.at[src_v.at[0]], rows0_v, sem0)

        def pair(j2, _):
            j = 2 * j2
            pltpu.async_copy(u_hbm.at[src_v.at[j + 1]], rows1_v, sem1)
            pltpu.make_async_copy(u_hbm.at[src_v.at[j]], rows0_v, sem0).wait()
            compute(rows0_v, msg0_v, j)

            @pl.when(j2 > 0)
            def _():
                pltpu.make_async_copy(msg0_v, acc_sh.at[dst_v.at[j]],
                                      sem2).wait()
            pltpu.async_copy(msg0_v, acc_sh.at[dst_v.at[j]], sem2, add=True)

            @pl.when(j2 + 1 < nch // 2)
            def _():
                pltpu.async_copy(u_hbm.at[src_v.at[j + 2]], rows0_v, sem0)
            pltpu.make_async_copy(u_hbm.at[src_v.at[j + 1]], rows1_v,
                                  sem1).wait()
            compute(rows1_v, msg1_v, j + 1)

            @pl.when(j2 > 0)
            def _():
                pltpu.make_async_copy(msg1_v, acc_sh.at[dst_v.at[j + 1]],
                                      sem3).wait()
            pltpu.async_copy(msg1_v, acc_sh.at[dst_v.at[j + 1]], sem3,
                             add=True)
            return 0
        lax.fori_loop(0, nch // 2, pair, 0)
        pltpu.make_async_copy(msg0_v, acc_sh.at[dst_v.at[0]], sem2).wait()
        pltpu.make_async_copy(msg1_v, acc_sh.at[dst_v.at[1]], sem3).wait()

        plsc.subcore_barrier()
        pltpu.sync_copy(acc_sh.at[pl.ds(s * npt, npt)], nbuf_v)
        pltpu.sync_copy(nbuf_v, out_hbm.at[c].at[pl.ds(s * npt, npt)])
        if tail:
            @pl.when(s == 0)
            def _():
                pltpu.sync_copy(acc_sh.at[pl.ds(NS * npt, tail)],
                                nbuf_v.at[pl.ds(0, tail)])
                pltpu.sync_copy(nbuf_v.at[pl.ds(0, tail)],
                                out_hbm.at[c].at[pl.ds(NS * npt, tail)])


def _make_sc_aggregate(n, h, ed, ep, true_e):
    epw = ep // NW
    k = 128
    nch = epw // k
    assert epw * NW == ep and nch * k == epw and nch % 2 == 0
    uw = (ed + 1) * h
    mesh = plsc.VectorSubcoreMesh(core_axis_name="c", subcore_axis_name="s",
                                  num_cores=NC, num_subcores=NS)
    return pl.kernel(
        functools.partial(_sc_edge_body, n, h, epw, nch, k),
        out_type=jax.ShapeDtypeStruct((NC, n, h), jnp.float32),
        mesh=mesh,
        scratch_types=[
            pltpu.VMEM((nch, k), jnp.int32),
            pltpu.VMEM((nch, k), jnp.int32),
            pltpu.VMEM((ed, epw // LANES, LANES), jnp.float32),
            pltpu.VMEM((k, uw), jnp.float32),
            pltpu.VMEM((k, uw), jnp.float32),
            pltpu.VMEM((k, h), jnp.float32),
            pltpu.VMEM((k, h), jnp.float32),
            pltpu.VMEM(((n // NS) // 8 * 8, h), jnp.float32),
            pltpu.VMEM_SHARED((n + max(ep - true_e, 8), h), jnp.float32),
            pltpu.SemaphoreType.DMA,
            pltpu.SemaphoreType.DMA,
            pltpu.SemaphoreType.DMA,
            pltpu.SemaphoreType.DMA,
        ],
        compiler_params=pltpu.CompilerParams(use_tc_tiling_on_sc=False),
    )


def _t1_body(x_ref, wemb_ref, bemb_ref, wcat_ref, h_ref, u_ref):
    hv = jnp.maximum(
        jnp.dot(x_ref[...], wemb_ref[...],
                preferred_element_type=jnp.float32) + bemb_ref[...], 0.0)
    h_ref[...] = hv
    u_ref[...] = jnp.dot(hv, wcat_ref[...], preferred_element_type=jnp.float32)


def _t2_body(acc_ref, h_ref, root_ref, bias_ref, wcat_ref, h_out_ref, u_ref):
    hv = jnp.maximum(
        acc_ref[0] + acc_ref[1]
        + jnp.dot(h_ref[...], root_ref[...],
                  preferred_element_type=jnp.float32) + bias_ref[...], 0.0)
    h_out_ref[...] = hv
    u_ref[...] = jnp.dot(hv, wcat_ref[...], preferred_element_type=jnp.float32)


def _t3_body(g, acc_ref, h_ref, root_ref, bias_ref, batch_ref, w1_ref,
             b1_ref, out_ref, pooled_ref):
    i = pl.program_id(0)
    hv = jnp.maximum(
        acc_ref[0] + acc_ref[1]
        + jnp.dot(h_ref[...], root_ref[...],
                  preferred_element_type=jnp.float32) + bias_ref[...], 0.0)
    bn = h_ref.shape[0]
    gid = lax.broadcasted_iota(jnp.int32, (g, bn), 0)
    onehot = (gid == batch_ref[0]).astype(jnp.float32)
    contrib = jnp.dot(onehot, hv, preferred_element_type=jnp.float32)

    @pl.when(i == 0)
    def _():
        pooled_ref[...] = jnp.zeros_like(pooled_ref)

    pooled_ref[...] += contrib

    @pl.when(i == pl.num_programs(0) - 1)
    def _():
        out_ref[...] = jnp.dot(pooled_ref[...], w1_ref[...],
                               preferred_element_type=jnp.float32) + b1_ref[...]


def _wcat(nnW, nnb, ed, h):
    w = nnW.reshape(ed, h, h).transpose(1, 0, 2).reshape(h, ed * h)
    return jnp.concatenate([w, nnb.reshape(h, h)], axis=1)


def kernel(x, edge_index, edge_attr, batch, W_emb, b_emb, nnW1, nnb1, root1,
           bias1, nnW2, nnb2, root2, bias2, nnW3, nnb3, root3, bias3, W1, b1):
    n, d = x.shape
    h = W_emb.shape[1]
    e = edge_index.shape[1]
    ed = edge_attr.shape[1]
    o = W1.shape[1]
    g = 64
    uw = (ed + 1) * h
    bn = 1000
    nblk = n // bn
    k = 128
    ep = -(-e // (NW * 2 * k)) * (NW * 2 * k)
    pad = ep - e

    src3 = jnp.concatenate(
        [edge_index[0],
         jnp.arange(pad, dtype=jnp.int32) % n]).reshape(ep // k, k)
    dst3 = jnp.concatenate(
        [edge_index[1], n + jnp.arange(pad, dtype=jnp.int32)]).reshape(
            ep // k, k)
    ea_t = jnp.pad(edge_attr.T, ((0, 0), (0, pad))).reshape(
        ed, ep // LANES, LANES)
    batch3 = batch.reshape(nblk, 1, bn)

    wc1 = _wcat(nnW1, nnb1, ed, h)
    wc2 = _wcat(nnW2, nnb2, ed, h)
    wc3 = _wcat(nnW3, nnb3, ed, h)

    sc_aggr = _make_sc_aggregate(n, h, ed, ep, e)

    full = lambda shape: pl.BlockSpec(shape, lambda i: tuple(0 for _ in shape))
    rows = lambda w: pl.BlockSpec((bn, w), lambda i: (i, 0))

    t1 = pl.pallas_call(
        _t1_body,
        grid=(nblk,),
        in_specs=[rows(d), full((d, h)), full((1, h)), full((h, uw))],
        out_specs=[rows(h), rows(uw)],
        out_shape=[jax.ShapeDtypeStruct((n, h), jnp.float32),
                   jax.ShapeDtypeStruct((n, uw), jnp.float32)],
    )

    t2 = pl.pallas_call(
        _t2_body,
        grid=(nblk,),
        in_specs=[pl.BlockSpec((NC, bn, h), lambda i: (0, i, 0)),
                  rows(h), full((h, h)), full((1, h)), full((h, uw))],
        out_specs=[rows(h), rows(uw)],
        out_shape=[jax.ShapeDtypeStruct((n, h), jnp.float32),
                   jax.ShapeDtypeStruct((n, uw), jnp.float32)],
    )

    t3 = pl.pallas_call(
        functools.partial(_t3_body, g),
        grid=(nblk,),
        in_specs=[pl.BlockSpec((NC, bn, h), lambda i: (0, i, 0)),
                  rows(h), full((h, h)), full((1, h)),
                  pl.BlockSpec((1, 1, bn), lambda i: (i, 0, 0)),
                  full((h, o)), full((1, o))],
        out_specs=pl.BlockSpec((g, o), lambda i: (0, 0)),
        out_shape=jax.ShapeDtypeStruct((g, o), jnp.float32),
        scratch_shapes=[pltpu.VMEM((g, h), jnp.float32)],
    )

    b_emb2 = b_emb.reshape(1, h)
    h0, u1 = t1(x, W_emb, b_emb2, wc1)
    acc1 = sc_aggr(u1, src3, dst3, ea_t)
    h1, u2 = t2(acc1, h0, root1, bias1.reshape(1, h), wc2)
    acc2 = sc_aggr(u2, src3, dst3, ea_t)
    h2, u3 = t2(acc2, h1, root2, bias2.reshape(1, h), wc3)
    acc3 = sc_aggr(u3, src3, dst3, ea_t)
    return t3(acc3, h2, root3, bias3.reshape(1, h), batch3, W1,
              b1.reshape(1, o))
